# baseline (speedup 1.0000x reference)
import functools

import jax
import jax.numpy as jnp
from jax import lax
from jax.experimental import pallas as pl
from jax.experimental.pallas import tpu as pltpu
from jax.experimental.pallas import tpu_sc as plsc

ROWS = 8
NBUF = 4


def _make_sc_gather(V, D, B):
    info = plsc.get_sparse_core_info()
    NC, NS = info.num_cores, info.num_subcores
    NW = NC * NS
    b_per_w = B // NW
    mesh = plsc.VectorSubcoreMesh(core_axis_name="c", subcore_axis_name="s")

    @functools.partial(
        pl.kernel,
        mesh=mesh,
        out_type=jax.ShapeDtypeStruct((B, D), jnp.float32),
        scratch_types=[
            pltpu.VMEM((b_per_w,), jnp.int32),
            pltpu.VMEM((b_per_w, D), jnp.float32),
            pltpu.SemaphoreType.DMA,
        ],
        compiler_params=pltpu.CompilerParams(use_tc_tiling_on_sc=False),
    )
    def gather_kernel(idx_hbm, table_hbm, out_hbm, idx_v, rows_v, sem):
        wid = lax.axis_index("s") * NC + lax.axis_index("c")
        base = wid * b_per_w
        pltpu.sync_copy(idx_hbm.at[pl.ds(base, b_per_w)], idx_v)
        pltpu.async_copy(table_hbm.at[idx_v], rows_v, sem).wait()
        pltpu.sync_copy(rows_v, out_hbm.at[pl.ds(base, b_per_w)])

    return gather_kernel


def _make_matmul(B, D, V):
    nb = B // ROWS

    def body(e_ref, wt_ref, b_ref, o_ref, ring, sems):
        i = pl.program_id(0)
        acc = lax.dot_general(
            e_ref[...], wt_ref[...],
            dimension_numbers=(((1,), (0,)), ((), ())),
            preferred_element_type=jnp.float32,
        ) + b_ref[...]
        for k in range(NBUF):
            @pl.when(lax.rem(i, NBUF) == k)
            def _(k=k):
                @pl.when(i >= NBUF)
                def _():
                    pltpu.make_async_copy(
                        ring.at[k], o_ref.at[0], sems.at[k]
                    ).wait()
                ring[k] = acc
                pltpu.make_async_copy(
                    ring.at[k], o_ref.at[i], sems.at[k]
                ).start()
        @pl.when(i == nb - 1)
        def _():
            for k in range(NBUF):
                pltpu.make_async_copy(
                    ring.at[k], o_ref.at[0], sems.at[k]
                ).wait()

    return pl.pallas_call(
        body,
        grid=(nb,),
        in_specs=[
            pl.BlockSpec((ROWS, D), lambda i: (i, 0)),
            pl.BlockSpec((D, V), lambda i: (0, 0)),
            pl.BlockSpec((1, V), lambda i: (0, 0)),
        ],
        out_specs=pl.BlockSpec(memory_space=pl.ANY),
        out_shape=jax.ShapeDtypeStruct((nb, ROWS, V), jnp.float32),
        scratch_shapes=[
            pltpu.VMEM((NBUF, ROWS, V), jnp.float32),
            pltpu.SemaphoreType.DMA((NBUF,)),
        ],
    )


def kernel(center_ids, embed, W, b):
    B, = center_ids.shape
    V, D = W.shape
    ids = center_ids.astype(jnp.int32)

    embeds = _make_sc_gather(V, D, B)(ids, embed)

    b2 = b.reshape(1, V)
    WT = W.T
    out3 = _make_matmul(B, D, V)(embeds, WT, b2)
    return out3.reshape(B, V)


# speedup vs baseline: 1.0573x; 1.0573x over previous
import functools

import jax
import jax.numpy as jnp
from jax import lax
from jax.experimental import pallas as pl
from jax.experimental.pallas import tpu as pltpu
from jax.experimental.pallas import tpu_sc as plsc

ROWS = 16
NBUF = 4


def _make_sc_gather(V, D, B):
    info = plsc.get_sparse_core_info()
    NC, NS = info.num_cores, info.num_subcores
    NW = NC * NS
    b_per_w = B // NW
    mesh = plsc.VectorSubcoreMesh(core_axis_name="c", subcore_axis_name="s")

    @functools.partial(
        pl.kernel,
        mesh=mesh,
        out_type=jax.ShapeDtypeStruct((B, D), jnp.float32),
        scratch_types=[
            pltpu.VMEM((b_per_w,), jnp.int32),
            pltpu.VMEM((b_per_w, D), jnp.float32),
            pltpu.SemaphoreType.DMA,
        ],
        compiler_params=pltpu.CompilerParams(use_tc_tiling_on_sc=False),
    )
    def gather_kernel(idx_hbm, table_hbm, out_hbm, idx_v, rows_v, sem):
        wid = lax.axis_index("s") * NC + lax.axis_index("c")
        base = wid * b_per_w
        pltpu.sync_copy(idx_hbm.at[pl.ds(base, b_per_w)], idx_v)
        pltpu.async_copy(table_hbm.at[idx_v], rows_v, sem).wait()
        pltpu.sync_copy(rows_v, out_hbm.at[pl.ds(base, b_per_w)])

    return gather_kernel


def _make_matmul(B, D, V):
    nb = B // ROWS

    def body(e_ref, wt_ref, b_ref, o_ref, ring, sems):
        i = pl.program_id(0)
        acc = lax.dot_general(
            e_ref[...], wt_ref[...],
            dimension_numbers=(((1,), (0,)), ((), ())),
            preferred_element_type=jnp.float32,
        ) + b_ref[...]
        for k in range(NBUF):
            @pl.when(lax.rem(i, NBUF) == k)
            def _(k=k):
                @pl.when(i >= NBUF)
                def _():
                    pltpu.make_async_copy(
                        ring.at[k], o_ref.at[0], sems.at[k]
                    ).wait()
                ring[k] = acc
                pltpu.make_async_copy(
                    ring.at[k], o_ref.at[i], sems.at[k]
                ).start()
        @pl.when(i == nb - 1)
        def _():
            for k in range(NBUF):
                pltpu.make_async_copy(
                    ring.at[k], o_ref.at[0], sems.at[k]
                ).wait()

    return pl.pallas_call(
        body,
        grid=(nb,),
        in_specs=[
            pl.BlockSpec((ROWS, D), lambda i: (i, 0)),
            pl.BlockSpec((D, V), lambda i: (0, 0)),
            pl.BlockSpec((1, V), lambda i: (0, 0)),
        ],
        out_specs=pl.BlockSpec(memory_space=pl.ANY),
        out_shape=jax.ShapeDtypeStruct((nb, ROWS, V), jnp.float32),
        scratch_shapes=[
            pltpu.VMEM((NBUF, ROWS, V), jnp.float32),
            pltpu.SemaphoreType.DMA((NBUF,)),
        ],
    )


def kernel(center_ids, embed, W, b):
    B, = center_ids.shape
    V, D = W.shape
    ids = center_ids.astype(jnp.int32)

    embeds = _make_sc_gather(V, D, B)(ids, embed)

    b2 = b.reshape(1, V)
    WT = W.T
    out3 = _make_matmul(B, D, V)(embeds, WT, b2)
    return out3.reshape(B, V)


# speedup vs baseline: 1.0629x; 1.0053x over previous
import functools

import jax
import jax.numpy as jnp
from jax import lax
from jax.experimental import pallas as pl
from jax.experimental.pallas import tpu as pltpu
from jax.experimental.pallas import tpu_sc as plsc

ROWS = 16
NBUF = 2


def _make_sc_gather(V, D, B):
    info = plsc.get_sparse_core_info()
    NC, NS = info.num_cores, info.num_subcores
    NW = NC * NS
    b_per_w = B // NW
    mesh = plsc.VectorSubcoreMesh(core_axis_name="c", subcore_axis_name="s")

    @functools.partial(
        pl.kernel,
        mesh=mesh,
        out_type=jax.ShapeDtypeStruct((B, D), jnp.float32),
        scratch_types=[
            pltpu.VMEM((b_per_w,), jnp.int32),
            pltpu.VMEM((b_per_w, D), jnp.float32),
            pltpu.SemaphoreType.DMA,
        ],
        compiler_params=pltpu.CompilerParams(use_tc_tiling_on_sc=False),
    )
    def gather_kernel(idx_hbm, table_hbm, out_hbm, idx_v, rows_v, sem):
        wid = lax.axis_index("s") * NC + lax.axis_index("c")
        base = wid * b_per_w
        pltpu.sync_copy(idx_hbm.at[pl.ds(base, b_per_w)], idx_v)
        pltpu.async_copy(table_hbm.at[idx_v], rows_v, sem).wait()
        pltpu.sync_copy(rows_v, out_hbm.at[pl.ds(base, b_per_w)])

    return gather_kernel


def _make_matmul(B, D, V):
    nb = B // ROWS

    def body(e_ref, wt_ref, b_ref, o_ref, ring, sems):
        i = pl.program_id(0)
        acc = lax.dot_general(
            e_ref[...], wt_ref[...],
            dimension_numbers=(((1,), (0,)), ((), ())),
            preferred_element_type=jnp.float32,
        ) + b_ref[...]
        for k in range(NBUF):
            @pl.when(lax.rem(i, NBUF) == k)
            def _(k=k):
                @pl.when(i >= NBUF)
                def _():
                    for r in range(ROWS):
                        pltpu.make_async_copy(
                            ring.at[k, pl.ds(r, 1)], o_ref.at[0, pl.ds(r, 1)],
                            sems.at[k],
                        ).wait()
                ring[k] = acc
                for r in range(ROWS):
                    pltpu.make_async_copy(
                        ring.at[k, pl.ds(r, 1)], o_ref.at[i, pl.ds(r, 1)],
                        sems.at[k],
                    ).start()
        @pl.when(i == nb - 1)
        def _():
            for k in range(NBUF):
                for r in range(ROWS):
                    pltpu.make_async_copy(
                        ring.at[k, pl.ds(r, 1)], o_ref.at[0, pl.ds(r, 1)],
                        sems.at[k],
                    ).wait()

    return pl.pallas_call(
        body,
        grid=(nb,),
        in_specs=[
            pl.BlockSpec((ROWS, D), lambda i: (i, 0)),
            pl.BlockSpec((D, V), lambda i: (0, 0)),
            pl.BlockSpec((1, V), lambda i: (0, 0)),
        ],
        out_specs=pl.BlockSpec(memory_space=pl.ANY),
        out_shape=jax.ShapeDtypeStruct((nb, ROWS, V), jnp.float32),
        scratch_shapes=[
            pltpu.VMEM((NBUF, ROWS, V), jnp.float32),
            pltpu.SemaphoreType.DMA((NBUF,)),
        ],
    )


def kernel(center_ids, embed, W, b):
    B, = center_ids.shape
    V, D = W.shape
    ids = center_ids.astype(jnp.int32)

    embeds = _make_sc_gather(V, D, B)(ids, embed)

    b2 = b.reshape(1, V)
    WT = W.T
    out3 = _make_matmul(B, D, V)(embeds, WT, b2)
    return out3.reshape(B, V)
